# baseline (speedup 1.0000x reference)
import functools

import jax
import jax.numpy as jnp
from jax import lax
from jax.experimental import pallas as pl
from jax.experimental.pallas import tpu as pltpu
from jax.experimental.pallas import tpu_sc as plsc


def _rsqrt_vec(s):
    i = lax.bitcast_convert_type(s, jnp.int32)
    i = jnp.int32(0x5F3759DF) - lax.shift_right_logical(i, 1)
    y = lax.bitcast_convert_type(i, jnp.float32)
    for _ in range(2):
        y = y * (jnp.float32(1.5) - jnp.float32(0.5) * s * y * y)
    return y


def _allsum(p):
    lanes = jnp.arange(16, dtype=jnp.int32)
    for k in (1, 2, 4, 8):
        p = p + p.at[lanes ^ k].get(mode="promise_in_bounds")
    return p


def kernel(entity_embds, rel_embds):
    n, d = entity_embds.shape
    mesh = plsc.VectorSubcoreMesh(core_axis_name="c", subcore_axis_name="s")
    nw = mesh.num_cores * mesh.num_subcores
    chunk = 1000
    nchunks = n // chunk

    @functools.partial(
        pl.kernel,
        out_type=jax.ShapeDtypeStruct((n, d), jnp.float32),
        mesh=mesh,
        scratch_types=[pltpu.VMEM((chunk, d), jnp.float32)],
    )
    def body(ent_hbm, out_hbm, buf):
        wid = lax.axis_index("s") * mesh.num_cores + lax.axis_index("c")
        my_chunks = (nchunks - wid + nw - 1) // nw

        def do_chunk(k, carry):
            base = pl.multiple_of((wid + k * nw) * chunk, 8)
            pltpu.sync_copy(ent_hbm.at[pl.ds(base, chunk)], buf)

            @plsc.parallel_loop(0, chunk, unroll=4)
            def do_row(r):
                v0 = buf[r, pl.ds(0, 16)]
                v1 = buf[r, pl.ds(16, 16)]
                v2 = buf[r, pl.ds(32, 16)]
                v3 = buf[r, pl.ds(48, 16)]
                ssq = _allsum(v0 * v0 + v1 * v1 + v2 * v2 + v3 * v3)
                iv = _rsqrt_vec(ssq)
                iv = jnp.where(base + r == n - 1, jnp.float32(1.0), iv)
                buf[r, pl.ds(0, 16)] = v0 * iv
                buf[r, pl.ds(16, 16)] = v1 * iv
                buf[r, pl.ds(32, 16)] = v2 * iv
                buf[r, pl.ds(48, 16)] = v3 * iv
            pltpu.sync_copy(buf, out_hbm.at[pl.ds(base, chunk)])
            return carry

        lax.fori_loop(0, my_chunks, do_chunk, 0)

    out = body(entity_embds)
    return (out, rel_embds)


# speedup vs baseline: 1.0490x; 1.0490x over previous
import functools

import jax
import jax.numpy as jnp
from jax import lax
from jax.experimental import pallas as pl
from jax.experimental.pallas import tpu as pltpu
from jax.experimental.pallas import tpu_sc as plsc


def _rsqrt_vec(s):
    i = lax.bitcast_convert_type(s, jnp.int32)
    i = jnp.int32(0x5F3759DF) - lax.shift_right_logical(i, 1)
    y = lax.bitcast_convert_type(i, jnp.float32)
    for _ in range(2):
        y = y * (jnp.float32(1.5) - jnp.float32(0.5) * s * y * y)
    return y


def _allsum(p):
    lanes = jnp.arange(16, dtype=jnp.int32)
    for k in (1, 2, 4, 8):
        p = p + p.at[lanes ^ k].get(mode="promise_in_bounds")
    return p


def _normalize_rows(buf, nrows, last_exempt_row=None):
    @plsc.parallel_loop(0, nrows, unroll=4)
    def do_row(r):
        v0 = buf[r, pl.ds(0, 16)]
        v1 = buf[r, pl.ds(16, 16)]
        v2 = buf[r, pl.ds(32, 16)]
        v3 = buf[r, pl.ds(48, 16)]
        ssq = _allsum(v0 * v0 + v1 * v1 + v2 * v2 + v3 * v3)
        iv = _rsqrt_vec(ssq)
        if last_exempt_row is not None:
            iv = jnp.where(r == last_exempt_row, jnp.float32(1.0), iv)
        buf[r, pl.ds(0, 16)] = v0 * iv
        buf[r, pl.ds(16, 16)] = v1 * iv
        buf[r, pl.ds(32, 16)] = v2 * iv
        buf[r, pl.ds(48, 16)] = v3 * iv


def kernel(entity_embds, rel_embds):
    n, d = entity_embds.shape
    mesh = plsc.VectorSubcoreMesh(core_axis_name="c", subcore_axis_name="s")
    nw = mesh.num_cores * mesh.num_subcores
    chunk = 504
    nchunks = 1984
    tail = n - nchunks * chunk
    per_worker = nchunks // nw

    @functools.partial(
        pl.kernel,
        out_type=jax.ShapeDtypeStruct((n, d), jnp.float32),
        mesh=mesh,
        scratch_types=[
            pltpu.VMEM((chunk, d), jnp.float32),
            pltpu.VMEM((chunk, d), jnp.float32),
            pltpu.SemaphoreType.DMA,
            pltpu.SemaphoreType.DMA,
            pltpu.SemaphoreType.DMA,
            pltpu.SemaphoreType.DMA,
        ],
    )
    def body(ent_hbm, out_hbm, b0, b1, si0, si1, so0, so1):
        wid = lax.axis_index("s") * mesh.num_cores + lax.axis_index("c")
        bufs = (b0, b1)
        sin = (si0, si1)
        sout = (so0, so1)

        def base(k):
            return pl.multiple_of((wid + k * nw) * chunk, 8)

        def fire_in(k, b):
            pltpu.async_copy(ent_hbm.at[pl.ds(base(k), chunk)], bufs[b], sin[b])

        def wait_in(k, b):
            pltpu.make_async_copy(
                ent_hbm.at[pl.ds(base(k), chunk)], bufs[b], sin[b]).wait()

        def fire_out(k, b):
            pltpu.async_copy(bufs[b], out_hbm.at[pl.ds(base(k), chunk)], sout[b])

        def wait_out(k, b):
            pltpu.make_async_copy(
                bufs[b], out_hbm.at[pl.ds(base(k), chunk)], sout[b]).wait()

        fire_in(0, 0)

        def outer(kk, carry):
            for b in (0, 1):
                k = kk * 2 + b
                wait_in(k, b)

                @pl.when(k >= 1)
                def _():
                    wait_out(k - 1, 1 - b)

                @pl.when(k + 1 < per_worker)
                def _():
                    fire_in(k + 1, 1 - b)

                _normalize_rows(bufs[b], chunk)
                fire_out(k, b)
            return carry

        lax.fori_loop(0, per_worker // 2, outer, 0)
        wait_out(per_worker - 1, 1)

        @pl.when(wid == 0)
        def _():
            tbase = nchunks * chunk
            tbuf = b0.at[pl.ds(0, tail)]
            pltpu.sync_copy(ent_hbm.at[pl.ds(tbase, tail)], tbuf)
            _normalize_rows(tbuf, tail, last_exempt_row=tail - 1)
            pltpu.sync_copy(tbuf, out_hbm.at[pl.ds(tbase, tail)])

    out = body(entity_embds)
    return (out, rel_embds)


# speedup vs baseline: 1.1813x; 1.1261x over previous
import functools

import jax
import jax.numpy as jnp
from jax import lax
from jax.experimental import pallas as pl
from jax.experimental.pallas import tpu as pltpu
from jax.experimental.pallas import tpu_sc as plsc

_NBUF = 3


def _rsqrt_vec(s):
    i = lax.bitcast_convert_type(s, jnp.int32)
    i = jnp.int32(0x5F3759DF) - lax.shift_right_logical(i, 1)
    y = lax.bitcast_convert_type(i, jnp.float32)
    for _ in range(2):
        y = y * (jnp.float32(1.5) - jnp.float32(0.5) * s * y * y)
    return y


def _allsum(p):
    lanes = jnp.arange(16, dtype=jnp.int32)
    for k in (1, 2, 4, 8):
        p = p + p.at[lanes ^ k].get(mode="promise_in_bounds")
    return p


def _normalize_rows(buf, nrows, last_exempt_row=None):
    @plsc.parallel_loop(0, nrows, unroll=4)
    def do_row(r):
        v0 = buf[r, pl.ds(0, 16)]
        v1 = buf[r, pl.ds(16, 16)]
        v2 = buf[r, pl.ds(32, 16)]
        v3 = buf[r, pl.ds(48, 16)]
        ssq = _allsum(v0 * v0 + v1 * v1 + v2 * v2 + v3 * v3)
        iv = _rsqrt_vec(ssq)
        if last_exempt_row is not None:
            iv = jnp.where(r == last_exempt_row, jnp.float32(1.0), iv)
        buf[r, pl.ds(0, 16)] = v0 * iv
        buf[r, pl.ds(16, 16)] = v1 * iv
        buf[r, pl.ds(32, 16)] = v2 * iv
        buf[r, pl.ds(48, 16)] = v3 * iv


def kernel(entity_embds, rel_embds):
    n, d = entity_embds.shape
    mesh = plsc.VectorSubcoreMesh(core_axis_name="c", subcore_axis_name="s")
    nw = mesh.num_cores * mesh.num_subcores
    chunk = 336
    nchunks = 2976
    tail = n - nchunks * chunk
    per_worker = nchunks // nw

    @functools.partial(
        pl.kernel,
        out_type=jax.ShapeDtypeStruct((n, d), jnp.float32),
        mesh=mesh,
        scratch_types=(
            [pltpu.VMEM((chunk, d), jnp.float32)] * _NBUF
            + [pltpu.SemaphoreType.DMA] * (2 * _NBUF)
        ),
    )
    def body(ent_hbm, out_hbm, *scratch):
        bufs = scratch[:_NBUF]
        sin = scratch[_NBUF:2 * _NBUF]
        sout = scratch[2 * _NBUF:]
        wid = lax.axis_index("s") * mesh.num_cores + lax.axis_index("c")

        def base(k):
            return pl.multiple_of((wid + k * nw) * chunk, 8)

        def fire_in(k, b):
            pltpu.async_copy(ent_hbm.at[pl.ds(base(k), chunk)], bufs[b], sin[b])

        def wait_in(k, b):
            pltpu.make_async_copy(
                ent_hbm.at[pl.ds(base(k), chunk)], bufs[b], sin[b]).wait()

        def fire_out(k, b):
            pltpu.async_copy(bufs[b], out_hbm.at[pl.ds(base(k), chunk)], sout[b])

        def wait_out(k, b):
            pltpu.make_async_copy(
                bufs[b], out_hbm.at[pl.ds(base(k), chunk)], sout[b]).wait()

        for j in range(_NBUF - 1):
            fire_in(j, j)

        def outer(kk, carry):
            for b in range(_NBUF):
                k = kk * _NBUF + b
                wait_in(k, b)
                _normalize_rows(bufs[b], chunk)
                fire_out(k, b)
                nxt = (b + _NBUF - 1) % _NBUF

                @pl.when(k == 0)
                def _():
                    fire_in(_NBUF - 1, _NBUF - 1)

                @pl.when(jnp.logical_and(k >= 1, k + _NBUF - 1 < per_worker))
                def _():
                    wait_out(k - 1, nxt)
                    fire_in(k + _NBUF - 1, nxt)
            return carry

        lax.fori_loop(0, per_worker // _NBUF, outer, 0)
        for k in range(per_worker - _NBUF + 1, per_worker):
            wait_out(k - 1, (k - 1) % _NBUF)
        wait_out(per_worker - 1, (per_worker - 1) % _NBUF)

        @pl.when(wid == 0)
        def _():
            tbase = nchunks * chunk
            tbuf = bufs[0].at[pl.ds(0, tail)]
            pltpu.sync_copy(ent_hbm.at[pl.ds(tbase, tail)], tbuf)
            _normalize_rows(tbuf, tail, last_exempt_row=tail - 1)
            pltpu.sync_copy(tbuf, out_hbm.at[pl.ds(tbase, tail)])

    out = body(entity_embds)
    return (out, rel_embds)
